# baseline (speedup 1.0000x reference)
import jax
import jax.numpy as jnp
from jax import lax
from jax.experimental import pallas as pl
from jax.experimental.pallas import tpu as pltpu
from jax.experimental.pallas import tpu_sc as plsc

N = 10000
D = 256
E = 160000
NW = 32
NC = 2
E_PAD = 172032
EPW = E_PAD // NW
RNG = 320
CAP = 240
LST = NW * CAP
ACC_R = RNG + 8
PAD_DST = 1 << 20

_sc_mesh = plsc.VectorSubcoreMesh(core_axis_name="c", subcore_axis_name="s")
_sc_params = pltpu.CompilerParams(needs_layout_passes=False)


def _iota16():
    return lax.iota(jnp.int32, 16)


def _shift(x, delta):
    idx = jnp.clip(_iota16() + delta, 0, 15)[:, None]
    dn = lax.GatherDimensionNumbers(offset_dims=(), collapsed_slice_dims=(0,),
                                    start_index_map=(0,))
    return lax.gather(x, idx, dn, (1,),
                      mode=lax.GatherScatterMode.PROMISE_IN_BOUNDS)


def _runs(skey):
    it = _iota16()
    first = (_shift(skey, -1) != skey) | (it == 0)
    last = (_shift(skey, 1) != skey) | (it == 15)
    rank = it - plsc.cummax(jnp.where(first, it, 0))
    return last, rank


def _r1_body(src_hbm, dst_hbm, rsrc_hbm, rdst_hbm,
             src_v, dst_v, ssrc_v, sdst_v, cnt_v):
    cid = lax.axis_index("c")
    sid = lax.axis_index("s")
    w = sid * NC + cid
    pltpu.sync_copy(src_hbm.at[pl.ds(w * EPW, EPW)], src_v)
    pltpu.sync_copy(dst_hbm.at[pl.ds(w * EPW, EPW)], dst_v)

    def prefill(i, c):
        sdst_v[i // 15, pl.ds((i % 15) * 16, 16)] = jnp.full(
            (16,), PAD_DST, jnp.int32)
        return c
    lax.fori_loop(0, NW * (CAP // 16), prefill, 0)

    def zcnt(i, c):
        cnt_v[pl.ds(i * 16, 16)] = jnp.zeros((16,), jnp.int32)
        return c
    lax.fori_loop(0, 4, zcnt, 0)

    def step(i, c):
        dst16 = dst_v[pl.ds(i * 16, 16)]
        src16 = src_v[pl.ds(i * 16, 16)]
        real = dst16 < N
        d0 = jnp.where(real, dst16, 0)
        bucket = lax.shift_right_logical(d0 * 52429, 24)
        key = jnp.where(real, bucket, 63)
        packed = src16 + lax.shift_left(d0, 14)
        skey, sval = plsc.sort_key_val(key, packed)
        last, rank = _runs(skey)
        base = plsc.load_gather(cnt_v, [skey])
        pos = base + rank
        ok = (skey < NW) & (pos < CAP)
        plsc.store_scatter(ssrc_v, [skey, pos], sval & 16383, mask=ok)
        plsc.store_scatter(sdst_v, [skey, pos],
                           lax.shift_right_logical(sval, 14), mask=ok)
        plsc.store_scatter(cnt_v, [skey], pos + 1, mask=last & (skey < NW))
        return c
    lax.fori_loop(0, EPW // 16, step, 0)

    for b in range(NW):
        pltpu.sync_copy(ssrc_v.at[b], rsrc_hbm.at[b, w])
        pltpu.sync_copy(sdst_v.at[b], rdst_hbm.at[b, w])


_r1_kernel = pl.kernel(
    _r1_body,
    out_type=(jax.ShapeDtypeStruct((NW, NW, CAP), jnp.int32),
              jax.ShapeDtypeStruct((NW, NW, CAP), jnp.int32)),
    mesh=_sc_mesh,
    compiler_params=_sc_params,
    scratch_types=[
        pltpu.VMEM((EPW,), jnp.int32),
        pltpu.VMEM((EPW,), jnp.int32),
        pltpu.VMEM((NW, CAP), jnp.int32),
        pltpu.VMEM((NW, CAP), jnp.int32),
        pltpu.VMEM((64,), jnp.int32),
    ],
)


def _r2_body(rsrc_hbm, rdst_hbm, esrc_hbm, dloc_hbm, deg_hbm, meta_hbm,
             rs_v, rd_v, cnt_v, csr_v, es_v, dl_v, degf_v, meta_v):
    cid = lax.axis_index("c")
    sid = lax.axis_index("s")
    b = sid * NC + cid
    base_node = b * RNG
    pltpu.sync_copy(rsrc_hbm.at[b], rs_v)
    pltpu.sync_copy(rdst_hbm.at[b], rd_v)

    def zcnt(i, c):
        cnt_v[pl.ds(i * 16, 16)] = jnp.zeros((16,), jnp.int32)
        return c
    lax.fori_loop(0, 21, zcnt, 0)

    def keys_at(i):
        dst16 = rd_v[i // 15, pl.ds((i % 15) * 16, 16)]
        dloc = dst16 - base_node
        valid = (dloc >= 0) & (dloc < RNG)
        return jnp.where(valid, dloc, RNG)

    def count_step(i, c):
        key = keys_at(i)
        skey = plsc.sort_key_val(key, key)[0]
        last, rank = _runs(skey)
        base = plsc.load_gather(cnt_v, [skey])
        plsc.store_scatter(cnt_v, [skey], base + rank + 1, mask=last)
        return c
    lax.fori_loop(0, NW * (CAP // 16), count_step, 0)

    def scan_step(i, carry):
        v = cnt_v[pl.ds(i * 16, 16)]
        c = plsc.cumsum(v)
        csr_v[pl.ds(i * 16, 16)] = c - v + carry
        return carry + c[15]
    lax.fori_loop(0, 21, scan_step, jnp.int32(0))

    nreal = csr_v[pl.ds(RNG, 16)][0]
    meta_v[pl.ds(0, 16)] = jnp.full((16,), 1, jnp.int32) * nreal

    def deg_step(i, c):
        degf_v[pl.ds(i * 16, 16)] = cnt_v[pl.ds(i * 16, 16)].astype(jnp.float32)
        return c
    lax.fori_loop(0, RNG // 16, deg_step, 0)

    def place_step(i, c):
        key = keys_at(i)
        src16 = rs_v[i // 15, pl.ds((i % 15) * 16, 16)]
        skey, ssrc = plsc.sort_key_val(key, src16)
        last, rank = _runs(skey)
        base = plsc.load_gather(csr_v, [skey])
        pos = base + rank
        safe_src = jnp.where(skey < RNG, ssrc, (_iota16() * 571 + i) & 8191)
        plsc.store_scatter(es_v, [pos], safe_src)
        plsc.store_scatter(dl_v, [pos], skey)
        plsc.store_scatter(csr_v, [skey], pos + 1, mask=last)
        return c
    lax.fori_loop(0, NW * (CAP // 16), place_step, 0)

    pltpu.sync_copy(es_v, esrc_hbm.at[b])
    pltpu.sync_copy(dl_v, dloc_hbm.at[b])
    pltpu.sync_copy(degf_v, deg_hbm.at[pl.ds(base_node, RNG)])
    pltpu.sync_copy(meta_v, meta_hbm.at[b])


_r2_kernel = pl.kernel(
    _r2_body,
    out_type=(jax.ShapeDtypeStruct((NW, LST), jnp.int32),
              jax.ShapeDtypeStruct((NW, LST), jnp.int32),
              jax.ShapeDtypeStruct((NW * RNG,), jnp.float32),
              jax.ShapeDtypeStruct((NW, 16), jnp.int32)),
    mesh=_sc_mesh,
    compiler_params=_sc_params,
    scratch_types=[
        pltpu.VMEM((NW, CAP), jnp.int32),
        pltpu.VMEM((NW, CAP), jnp.int32),
        pltpu.VMEM((336,), jnp.int32),
        pltpu.VMEM((336,), jnp.int32),
        pltpu.VMEM((LST,), jnp.int32),
        pltpu.VMEM((LST,), jnp.int32),
        pltpu.VMEM((RNG,), jnp.float32),
        pltpu.VMEM((16,), jnp.int32),
    ],
)


CH = 48


def _hop_body(g_hbm, esrc_hbm, dloc_hbm, meta_hbm, zeros_hbm, s_hbm,
              es_v, dl_v, meta_v, acc_v, rows_a, rows_b, sem_a, sem_b):
    cid = lax.axis_index("c")
    sid = lax.axis_index("s")
    b = sid * NC + cid
    pltpu.sync_copy(esrc_hbm.at[b], es_v)
    pltpu.sync_copy(dloc_hbm.at[b], dl_v.at[pl.ds(0, LST)])
    pltpu.sync_copy(meta_hbm.at[b], meta_v)
    pltpu.sync_copy(zeros_hbm, acc_v)
    nreal = meta_v[pl.ds(0, 16)][0]
    nch = lax.div(nreal + (CH - 1), jnp.int32(CH))

    def start(k, rows_ref, sem):
        pltpu.async_copy(g_hbm.at[es_v.at[pl.ds(k * CH, CH)]], rows_ref, sem)

    def drain(rows_ref, sem):
        pltpu.make_async_copy(g_hbm.at[pl.ds(0, CH)], rows_ref, sem).wait()

    def edges(kbase, rows_ref, trip, carry):
        def edge(e, ec):
            cur, regs = ec
            d = dl_v[pl.ds(kbase * CH + e, 16)][0]
            eq = d == cur

            @pl.when(jnp.logical_not(eq))
            def _():
                for j in range(16):
                    acc_v[cur, pl.ds(j * 16, 16)] = regs[j]

            keep = jnp.where(eq, 1.0, 0.0).astype(jnp.float32)
            nregs = []
            for j in range(8):
                w = rows_ref[e, pl.ds(j * 16, 16)]
                lo = plsc.bitcast(lax.shift_left(w, 16), jnp.float32)
                hi = plsc.bitcast(w & jnp.int32(-65536), jnp.float32)
                nregs.append(regs[2 * j] * keep + lo)
                nregs.append(regs[2 * j + 1] * keep + hi)
            return (d, tuple(nregs))

        return plsc.parallel_loop(0, trip, 1, unroll=16, carry=carry)(edge)

    zreg = jnp.zeros((16,), jnp.float32)
    init = (jnp.int32(ACC_R - 1), tuple(zreg for _ in range(16)))

    @pl.when(nch > 0)
    def _():
        start(jnp.int32(0), rows_a, sem_a)

    @pl.when(nch > 1)
    def _():
        start(jnp.int32(1), rows_b, sem_b)

    def pair(kk, carry):
        k0 = kk * 2
        k1 = k0 + 1
        drain(rows_a, sem_a)
        carry = edges(k0, rows_a, jnp.int32(CH), carry)

        @pl.when(k0 + 2 < nch)
        def _():
            start(k0 + 2, rows_a, sem_a)

        @pl.when(k1 < nch)
        def _():
            drain(rows_b, sem_b)

        trip1 = jnp.where(k1 < nch, jnp.int32(CH), jnp.int32(0))
        carry = edges(k1, rows_b, trip1, carry)

        @pl.when(k1 + 2 < nch)
        def _():
            start(k1 + 2, rows_b, sem_b)

        return carry

    npair = lax.div(nch + 1, jnp.int32(2))
    cur, regs = lax.fori_loop(0, npair, pair, init)
    for j in range(16):
        acc_v[cur, pl.ds(j * 16, 16)] = regs[j]
    pltpu.sync_copy(acc_v.at[pl.ds(0, RNG)], s_hbm.at[pl.ds(b * RNG, RNG)])


_hop_kernel = pl.kernel(
    _hop_body,
    out_type=jax.ShapeDtypeStruct((NW * RNG, D), jnp.float32),
    mesh=_sc_mesh,
    compiler_params=_sc_params,
    scratch_types=[
        pltpu.VMEM((LST,), jnp.int32),
        pltpu.VMEM((LST + 16,), jnp.int32),
        pltpu.VMEM((16,), jnp.int32),
        pltpu.VMEM((ACC_R, D), jnp.float32),
        pltpu.VMEM((CH, D // 2), jnp.int32),
        pltpu.VMEM((CH, D // 2), jnp.int32),
        pltpu.SemaphoreType.DMA,
        pltpu.SemaphoreType.DMA,
    ],
)


R = 1000


def _scale_body(x_ref, dg_ref, o_ref):
    deg = dg_ref[...]
    o_ref[...] = (x_ref[...] * lax.rsqrt(deg)).astype(jnp.bfloat16)


def _mid_body(s_ref, dg_ref, o_ref):
    deg = dg_ref[...]
    o_ref[...] = (s_ref[...] / deg).astype(jnp.bfloat16)


def _out_body(s_ref, dg_ref, w_ref, b_ref, o_ref):
    deg = dg_ref[...]
    h = s_ref[...] * lax.rsqrt(deg)
    o_ref[...] = (jnp.dot(h, w_ref[...], preferred_element_type=jnp.float32)
                  + b_ref[...])


_row_spec = pl.BlockSpec((R, D), lambda i: (i, 0))
_deg_spec = pl.BlockSpec((R, 1), lambda i: (i, 0))

_scale_kernel = pl.pallas_call(
    _scale_body,
    grid=(N // R,),
    in_specs=[_row_spec, _deg_spec],
    out_specs=_row_spec,
    out_shape=jax.ShapeDtypeStruct((N, D), jnp.bfloat16),
)

_mid_kernel = pl.pallas_call(
    _mid_body,
    grid=(N // R,),
    in_specs=[_row_spec, _deg_spec],
    out_specs=_row_spec,
    out_shape=jax.ShapeDtypeStruct((N, D), jnp.bfloat16),
)

_out_kernel = pl.pallas_call(
    _out_body,
    grid=(N // R,),
    in_specs=[_row_spec, _deg_spec,
              pl.BlockSpec((D, D), lambda i: (0, 0)),
              pl.BlockSpec((1, D), lambda i: (0, 0))],
    out_specs=_row_spec,
    out_shape=jax.ShapeDtypeStruct((N, D), jnp.float32),
)


def kernel(x, edge_index, dst_node_ids, W, b):
    src = edge_index[0].astype(jnp.int32)
    dst = edge_index[1].astype(jnp.int32)
    loop = jnp.arange(N, dtype=jnp.int32)
    pad_n = E_PAD - E - N
    src_p = jnp.concatenate([src, loop, jnp.zeros((pad_n,), jnp.int32)])
    dst_p = jnp.concatenate([dst, loop, jnp.full((pad_n,), PAD_DST, jnp.int32)])
    zrows = jnp.zeros((ACC_R, D), jnp.float32)
    c = jnp.arange(D, dtype=jnp.int32)
    sigma = 32 * (c // 32) + 2 * (c % 16) + (c % 32) // 16
    sigma2 = sigma[sigma]
    Wp = W[sigma2, :]

    rsrc, rdst = _r1_kernel(src_p, dst_p)
    esrc, dloc, degf, meta = _r2_kernel(rsrc, rdst)
    deg2 = degf.reshape(NW * RNG, 1)

    def _as_i32(g):
        return lax.bitcast_convert_type(g.reshape(N, D // 2, 2), jnp.int32)

    g1 = _scale_kernel(x.astype(jnp.float32), deg2)
    s1 = _hop_kernel(_as_i32(g1), esrc, dloc, meta, zrows)
    g2 = _mid_kernel(s1, deg2)
    s2 = _hop_kernel(_as_i32(g2), esrc, dloc, meta, zrows)
    out = _out_kernel(s2, deg2, Wp, b.reshape(1, D))
    return (out, dst_node_ids)


# speedup vs baseline: 1.4713x; 1.4713x over previous
import jax
import jax.numpy as jnp
from jax import lax
from jax.experimental import pallas as pl
from jax.experimental.pallas import tpu as pltpu
from jax.experimental.pallas import tpu_sc as plsc

N = 10000
D = 256
E = 160000
NW = 32
NC = 2
E_PAD = 163840
EPW = E_PAD // NW
RNG = 320
CAP = 240
LST = NW * CAP
ACC_R = RNG + 8
PAD_DST = 1 << 20

_sc_mesh = plsc.VectorSubcoreMesh(core_axis_name="c", subcore_axis_name="s")
_sc_params = pltpu.CompilerParams(needs_layout_passes=False)


def _iota16():
    return lax.iota(jnp.int32, 16)


def _shift(x, delta):
    idx = jnp.clip(_iota16() + delta, 0, 15)[:, None]
    dn = lax.GatherDimensionNumbers(offset_dims=(), collapsed_slice_dims=(0,),
                                    start_index_map=(0,))
    return lax.gather(x, idx, dn, (1,),
                      mode=lax.GatherScatterMode.PROMISE_IN_BOUNDS)


def _runs(skey):
    it = _iota16()
    first = (_shift(skey, -1) != skey) | (it == 0)
    last = (_shift(skey, 1) != skey) | (it == 15)
    rank = it - plsc.cummax(jnp.where(first, it, 0))
    return last, rank


def _r1_body(src_hbm, dst_hbm, rsrc_hbm, rdst_hbm,
             src_v, dst_v, ssrc_v, sdst_v, cnt_v):
    cid = lax.axis_index("c")
    sid = lax.axis_index("s")
    w = sid * NC + cid
    pltpu.sync_copy(src_hbm.at[pl.ds(w * EPW, EPW)], src_v)
    pltpu.sync_copy(dst_hbm.at[pl.ds(w * EPW, EPW)], dst_v)

    def prefill(i, c):
        sdst_v[i // 15, pl.ds((i % 15) * 16, 16)] = jnp.full(
            (16,), PAD_DST, jnp.int32)
        return c
    lax.fori_loop(0, NW * (CAP // 16), prefill, 0)

    def zcnt(i, c):
        cnt_v[pl.ds(i * 16, 16)] = jnp.zeros((16,), jnp.int32)
        return c
    lax.fori_loop(0, 4, zcnt, 0)

    def step(i, c):
        dst16 = dst_v[pl.ds(i * 16, 16)]
        src16 = src_v[pl.ds(i * 16, 16)]
        real = dst16 < N
        d0 = jnp.where(real, dst16, 0)
        bucket = lax.shift_right_logical(d0 * 52429, 24)
        key = jnp.where(real, bucket, 63)
        packed = src16 + lax.shift_left(d0, 14)
        skey, sval = plsc.sort_key_val(key, packed)
        last, rank = _runs(skey)
        base = plsc.load_gather(cnt_v, [skey])
        pos = base + rank
        ok = (skey < NW) & (pos < CAP)
        plsc.store_scatter(ssrc_v, [skey, pos], sval & 16383, mask=ok)
        plsc.store_scatter(sdst_v, [skey, pos],
                           lax.shift_right_logical(sval, 14), mask=ok)
        plsc.store_scatter(cnt_v, [skey], pos + 1, mask=last & (skey < NW))
        return c
    lax.fori_loop(0, EPW // 16, step, 0)

    for b in range(NW):
        pltpu.sync_copy(ssrc_v.at[b], rsrc_hbm.at[b, w])
        pltpu.sync_copy(sdst_v.at[b], rdst_hbm.at[b, w])


_r1_kernel = pl.kernel(
    _r1_body,
    out_type=(jax.ShapeDtypeStruct((NW, NW, CAP), jnp.int32),
              jax.ShapeDtypeStruct((NW, NW, CAP), jnp.int32)),
    mesh=_sc_mesh,
    compiler_params=_sc_params,
    scratch_types=[
        pltpu.VMEM((EPW,), jnp.int32),
        pltpu.VMEM((EPW,), jnp.int32),
        pltpu.VMEM((NW, CAP), jnp.int32),
        pltpu.VMEM((NW, CAP), jnp.int32),
        pltpu.VMEM((64,), jnp.int32),
    ],
)


def _r2_body(rsrc_hbm, rdst_hbm, esrc_hbm, dloc_hbm, deg_hbm, meta_hbm,
             rs_v, rd_v, cnt_v, csr_v, es_v, dl_v, degf_v, meta_v):
    cid = lax.axis_index("c")
    sid = lax.axis_index("s")
    b = sid * NC + cid
    base_node = b * RNG
    pltpu.sync_copy(rsrc_hbm.at[b], rs_v)
    pltpu.sync_copy(rdst_hbm.at[b], rd_v)

    def zcnt(i, c):
        cnt_v[pl.ds(i * 16, 16)] = jnp.zeros((16,), jnp.int32)
        return c
    lax.fori_loop(0, 21, zcnt, 0)

    def keys_at(i):
        dst16 = rd_v[i // 15, pl.ds((i % 15) * 16, 16)]
        dloc = dst16 - base_node
        valid = (dloc >= 0) & (dloc < RNG)
        return jnp.where(valid, dloc, RNG)

    def count_step(i, c):
        key = keys_at(i)
        skey = plsc.sort_key_val(key, key)[0]
        last, rank = _runs(skey)
        base = plsc.load_gather(cnt_v, [skey])
        plsc.store_scatter(cnt_v, [skey], base + rank + 1, mask=last)
        return c
    lax.fori_loop(0, NW * (CAP // 16), count_step, 0)

    def scan_step(i, carry):
        v = cnt_v[pl.ds(i * 16, 16)]
        c = plsc.cumsum(v)
        csr_v[pl.ds(i * 16, 16)] = c - v + carry
        return carry + c[15]
    lax.fori_loop(0, 21, scan_step, jnp.int32(0))

    nreal = csr_v[pl.ds(RNG, 16)][0]
    meta_v[pl.ds(0, 16)] = jnp.full((16,), 1, jnp.int32) * nreal

    def deg_step(i, c):
        degf_v[pl.ds(i * 16, 16)] = cnt_v[pl.ds(i * 16, 16)].astype(jnp.float32)
        return c
    lax.fori_loop(0, RNG // 16, deg_step, 0)

    def place_step(i, c):
        key = keys_at(i)
        src16 = rs_v[i // 15, pl.ds((i % 15) * 16, 16)]
        skey, ssrc = plsc.sort_key_val(key, src16)
        last, rank = _runs(skey)
        base = plsc.load_gather(csr_v, [skey])
        pos = base + rank
        safe_src = jnp.where(skey < RNG, ssrc, (_iota16() * 571 + i) & 8191)
        plsc.store_scatter(es_v, [pos], safe_src)
        plsc.store_scatter(dl_v, [pos], skey)
        plsc.store_scatter(csr_v, [skey], pos + 1, mask=last)
        return c
    lax.fori_loop(0, NW * (CAP // 16), place_step, 0)

    pltpu.sync_copy(es_v, esrc_hbm.at[b])
    pltpu.sync_copy(dl_v, dloc_hbm.at[b])
    pltpu.sync_copy(degf_v, deg_hbm.at[pl.ds(base_node, RNG)])
    pltpu.sync_copy(meta_v, meta_hbm.at[b])


_r2_kernel = pl.kernel(
    _r2_body,
    out_type=(jax.ShapeDtypeStruct((NW, LST), jnp.int32),
              jax.ShapeDtypeStruct((NW, LST), jnp.int32),
              jax.ShapeDtypeStruct((NW * RNG,), jnp.float32),
              jax.ShapeDtypeStruct((NW, 16), jnp.int32)),
    mesh=_sc_mesh,
    compiler_params=_sc_params,
    scratch_types=[
        pltpu.VMEM((NW, CAP), jnp.int32),
        pltpu.VMEM((NW, CAP), jnp.int32),
        pltpu.VMEM((336,), jnp.int32),
        pltpu.VMEM((336,), jnp.int32),
        pltpu.VMEM((LST,), jnp.int32),
        pltpu.VMEM((LST,), jnp.int32),
        pltpu.VMEM((RNG,), jnp.float32),
        pltpu.VMEM((16,), jnp.int32),
    ],
)


CH = 48


def _hop_body(g_hbm, esrc_hbm, dloc_hbm, meta_hbm, zeros_hbm, s_hbm,
              es_v, dl_v, meta_v, acc_v, rows_a, rows_b, sem_a, sem_b):
    cid = lax.axis_index("c")
    sid = lax.axis_index("s")
    b = sid * NC + cid
    pltpu.sync_copy(esrc_hbm.at[b], es_v)
    pltpu.sync_copy(dloc_hbm.at[b], dl_v.at[pl.ds(0, LST)])
    pltpu.sync_copy(meta_hbm.at[b], meta_v)
    pltpu.sync_copy(zeros_hbm, acc_v)
    nreal = meta_v[pl.ds(0, 16)][0]
    nch = lax.div(nreal + (CH - 1), jnp.int32(CH))

    def start(k, rows_ref, sem):
        pltpu.async_copy(g_hbm.at[es_v.at[pl.ds(k * CH, CH)]], rows_ref, sem)

    def drain(rows_ref, sem):
        pltpu.make_async_copy(zeros_hbm.at[pl.ds(0, CH)], rows_ref, sem).wait()

    def edges(kbase, rows_ref, trip, carry):
        def edge(e, ec):
            cur, regs = ec
            d = dl_v[pl.ds(kbase * CH + e, 16)][0]
            eq = d == cur

            @pl.when(jnp.logical_not(eq))
            def _():
                for j in range(16):
                    acc_v[cur, pl.ds(j * 16, 16)] = regs[j]

            keep = jnp.where(eq, 1.0, 0.0).astype(jnp.float32)
            nregs = tuple(
                regs[j] * keep + rows_ref[e, pl.ds(j * 16, 16)]
                for j in range(16))
            return (d, nregs)

        return plsc.parallel_loop(0, trip, 1, unroll=16, carry=carry)(edge)

    zreg = jnp.zeros((16,), jnp.float32)
    init = (jnp.int32(ACC_R - 1), tuple(zreg for _ in range(16)))

    @pl.when(nch > 0)
    def _():
        start(jnp.int32(0), rows_a, sem_a)

    @pl.when(nch > 1)
    def _():
        start(jnp.int32(1), rows_b, sem_b)

    def pair(kk, carry):
        k0 = kk * 2
        k1 = k0 + 1
        drain(rows_a, sem_a)
        carry = edges(k0, rows_a, jnp.int32(CH), carry)

        @pl.when(k0 + 2 < nch)
        def _():
            start(k0 + 2, rows_a, sem_a)

        @pl.when(k1 < nch)
        def _():
            drain(rows_b, sem_b)

        trip1 = jnp.where(k1 < nch, jnp.int32(CH), jnp.int32(0))
        carry = edges(k1, rows_b, trip1, carry)

        @pl.when(k1 + 2 < nch)
        def _():
            start(k1 + 2, rows_b, sem_b)

        return carry

    npair = lax.div(nch + 1, jnp.int32(2))
    cur, regs = lax.fori_loop(0, npair, pair, init)
    for j in range(16):
        acc_v[cur, pl.ds(j * 16, 16)] = regs[j]
    pltpu.sync_copy(acc_v.at[pl.ds(0, RNG)], s_hbm.at[pl.ds(b * RNG, RNG)])


_hop_kernel = pl.kernel(
    _hop_body,
    out_type=jax.ShapeDtypeStruct((NW * RNG, D), jnp.float32),
    mesh=_sc_mesh,
    compiler_params=_sc_params,
    scratch_types=[
        pltpu.VMEM((LST,), jnp.int32),
        pltpu.VMEM((LST + 16,), jnp.int32),
        pltpu.VMEM((16,), jnp.int32),
        pltpu.VMEM((ACC_R, D), jnp.float32),
        pltpu.VMEM((CH, D), jnp.float32),
        pltpu.VMEM((CH, D), jnp.float32),
        pltpu.SemaphoreType.DMA,
        pltpu.SemaphoreType.DMA,
    ],
)


R = 1000


def _scale_body(x_ref, dg_ref, o_ref):
    deg = dg_ref[...] + 1.0
    o_ref[...] = x_ref[...] * lax.rsqrt(deg)


def _mid_body(s_ref, g_ref, dg_ref, o_ref):
    deg = dg_ref[...] + 1.0
    o_ref[...] = (s_ref[...] + g_ref[...]) / deg


def _out_body(s_ref, g_ref, dg_ref, w_ref, b_ref, o_ref):
    deg = dg_ref[...] + 1.0
    h = (s_ref[...] + g_ref[...]) * lax.rsqrt(deg)
    o_ref[...] = (jnp.dot(h, w_ref[...], preferred_element_type=jnp.float32)
                  + b_ref[...])


_row_spec = pl.BlockSpec((R, D), lambda i: (i, 0))
_deg_spec = pl.BlockSpec((R, 1), lambda i: (i, 0))

_scale_kernel = pl.pallas_call(
    _scale_body,
    grid=(N // R,),
    in_specs=[_row_spec, _deg_spec],
    out_specs=_row_spec,
    out_shape=jax.ShapeDtypeStruct((N, D), jnp.float32),
)

_mid_kernel = pl.pallas_call(
    _mid_body,
    grid=(N // R,),
    in_specs=[_row_spec, _row_spec, _deg_spec],
    out_specs=_row_spec,
    out_shape=jax.ShapeDtypeStruct((N, D), jnp.float32),
)

_out_kernel = pl.pallas_call(
    _out_body,
    grid=(N // R,),
    in_specs=[_row_spec, _row_spec, _deg_spec,
              pl.BlockSpec((D, D), lambda i: (0, 0)),
              pl.BlockSpec((1, D), lambda i: (0, 0))],
    out_specs=_row_spec,
    out_shape=jax.ShapeDtypeStruct((N, D), jnp.float32),
)


def kernel(x, edge_index, dst_node_ids, W, b):
    src = edge_index[0].astype(jnp.int32)
    dst = edge_index[1].astype(jnp.int32)
    pad_n = E_PAD - E
    src_p = jnp.concatenate([src, jnp.zeros((pad_n,), jnp.int32)])
    dst_p = jnp.concatenate([dst, jnp.full((pad_n,), PAD_DST, jnp.int32)])
    zrows = jnp.zeros((ACC_R, D), jnp.float32)

    rsrc, rdst = _r1_kernel(src_p, dst_p)
    esrc, dloc, degf, meta = _r2_kernel(rsrc, rdst)
    deg2 = degf.reshape(NW * RNG, 1)

    g1 = _scale_kernel(x.astype(jnp.float32), deg2)
    s1 = _hop_kernel(g1, esrc, dloc, meta, zrows)
    g2 = _mid_kernel(s1, g1, deg2)
    s2 = _hop_kernel(g2, esrc, dloc, meta, zrows)
    out = _out_kernel(s2, g2, deg2, W, b.reshape(1, D))
    return (out, dst_node_ids)
